# baseline (speedup 1.0000x reference)
import functools

import jax
import jax.numpy as jnp
from jax import lax
from jax.experimental import pallas as pl
from jax.experimental.pallas import tpu as pltpu
from jax.experimental.pallas import tpu_sc as plsc

N_TOK = 8192
D_MODEL = 2048
N_EXP = 64
LANES = 16
BLK = 128
N_WORKERS = 32
N_BLOCKS = N_TOK // BLK
TOK_BLK = 1024


def _logits_body(w_ref, hs_ref, out_ref):
    lg = lax.dot_general(
        w_ref[...], hs_ref[...],
        dimension_numbers=(((1,), (1,)), ((), ())),
        preferred_element_type=jnp.float32,
    )
    for j in range(TOK_BLK // BLK):
        out_ref[j, :, :] = lg[:, j * BLK:(j + 1) * BLK]


def _compute_logits_blocked(hidden_states, weight):
    return pl.pallas_call(
        _logits_body,
        grid=(N_TOK // TOK_BLK,),
        in_specs=[
            pl.BlockSpec((N_EXP, D_MODEL), lambda i: (0, 0)),
            pl.BlockSpec((TOK_BLK, D_MODEL), lambda i: (i, 0)),
        ],
        out_specs=pl.BlockSpec((TOK_BLK // BLK, N_EXP, BLK), lambda i: (i, 0, 0)),
        out_shape=jax.ShapeDtypeStruct((N_BLOCKS, N_EXP, BLK), jnp.float32),
    )(weight, hidden_states)


_SC_MESH = plsc.VectorSubcoreMesh(core_axis_name="c", subcore_axis_name="s")


def _top2_stream(ltile, off):
    m1 = ltile[0, 0, pl.ds(off, LANES)]
    i1 = jnp.zeros((LANES,), jnp.int32)
    m2 = jnp.full((LANES,), -jnp.inf, jnp.float32)
    i2 = i1
    for e in range(1, N_EXP):
        v = ltile[0, e, pl.ds(off, LANES)]
        ev = jnp.full((LANES,), e, jnp.int32)
        gt1 = v > m1
        gt2 = v > m2
        nm2 = jnp.where(gt1, m1, jnp.where(gt2, v, m2))
        ni2 = jnp.where(gt1, i1, jnp.where(gt2, ev, i2))
        m1 = jnp.where(gt1, v, m1)
        i1 = jnp.where(gt1, ev, i1)
        m2 = nm2
        i2 = ni2
    return m1, i1, m2, i2


@functools.partial(
    pl.kernel,
    out_type=(
        jax.ShapeDtypeStruct((2 * N_TOK,), jnp.int32),
        jax.ShapeDtypeStruct((2 * N_TOK,), jnp.float32),
    ),
    mesh=_SC_MESH,
    compiler_params=pltpu.CompilerParams(needs_layout_passes=False),
    scratch_types=[
        pltpu.VMEM((1, N_EXP, BLK), jnp.float32),
        pltpu.VMEM((2 * BLK,), jnp.int32),
        pltpu.VMEM((2 * BLK,), jnp.float32),
    ],
)
def _route_sc(logits_hbm, idx_hbm, w_hbm, ltile, idxv, wv):
    wid = lax.axis_index("s") * 2 + lax.axis_index("c")

    lanes = lax.iota(jnp.int32, LANES)

    def emit(tok, m1, i1, m2, i2):
        ed = jnp.exp(m2 - m1)
        s = ed + 1.0
        tok2 = tok * 2
        plsc.store_scatter(idxv, [tok2], i1)
        plsc.store_scatter(idxv, [tok2 + 1], i2)
        plsc.store_scatter(wv, [tok2], 1.0 / s)
        plsc.store_scatter(wv, [tok2 + 1], ed / s)

    def per_pair(p, carry):
        off_a = p * (2 * LANES)
        off_b = off_a + LANES
        ra = _top2_stream(ltile, off_a)
        rb = _top2_stream(ltile, off_b)
        emit(off_a + lanes, *ra)
        emit(off_b + lanes, *rb)
        return carry

    for k in range(N_BLOCKS // N_WORKERS):
        blk = wid + k * N_WORKERS
        pltpu.sync_copy(logits_hbm.at[pl.ds(blk, 1)], ltile)
        lax.fori_loop(0, BLK // (2 * LANES), per_pair, 0)
        pltpu.sync_copy(idxv, idx_hbm.at[pl.ds(blk * 2 * BLK, 2 * BLK)])
        pltpu.sync_copy(wv, w_hbm.at[pl.ds(blk * 2 * BLK, 2 * BLK)])


def kernel(hidden_states, weight):
    logits_blocked = _compute_logits_blocked(hidden_states, weight)
    idx_flat, w_flat = _route_sc(logits_blocked)
    return idx_flat.reshape(N_TOK, 2), w_flat.reshape(N_TOK, 2)


# speedup vs baseline: 1.0967x; 1.0967x over previous
import functools

import jax
import jax.numpy as jnp
from jax import lax
from jax.experimental import pallas as pl
from jax.experimental.pallas import tpu as pltpu
from jax.experimental.pallas import tpu_sc as plsc

N_TOK = 8192
D_MODEL = 2048
N_EXP = 64
LANES = 16
BLK = 128
N_WORKERS = 32
N_BLOCKS = N_TOK // BLK
N_CHUNKS = 2
CHUNK = N_TOK // N_CHUNKS
TOK_BLK = 1024


def _logits_body(w_ref, hs_ref, out_ref):
    lg = lax.dot_general(
        w_ref[...], hs_ref[...],
        dimension_numbers=(((1,), (1,)), ((), ())),
        preferred_element_type=jnp.float32,
    )
    for j in range(TOK_BLK // BLK):
        out_ref[j, :, :] = lg[:, j * BLK:(j + 1) * BLK]


def _compute_logits_blocked(hidden_states, weight, chunk):
    blk0 = chunk * (CHUNK // TOK_BLK)
    return pl.pallas_call(
        _logits_body,
        grid=(CHUNK // TOK_BLK,),
        in_specs=[
            pl.BlockSpec((N_EXP, D_MODEL), lambda i: (0, 0)),
            pl.BlockSpec((TOK_BLK, D_MODEL), lambda i: (blk0 + i, 0)),
        ],
        out_specs=pl.BlockSpec((TOK_BLK // BLK, N_EXP, BLK), lambda i: (i, 0, 0)),
        out_shape=jax.ShapeDtypeStruct((CHUNK // BLK, N_EXP, BLK), jnp.float32),
    )(weight, hidden_states)


_SC_MESH = plsc.VectorSubcoreMesh(core_axis_name="c", subcore_axis_name="s")


def _top2_stream(ltile, off):
    m1 = ltile[0, 0, pl.ds(off, LANES)]
    i1 = jnp.zeros((LANES,), jnp.int32)
    m2 = jnp.full((LANES,), -jnp.inf, jnp.float32)
    i2 = i1
    for e in range(1, N_EXP):
        v = ltile[0, e, pl.ds(off, LANES)]
        ev = jnp.full((LANES,), e, jnp.int32)
        gt1 = v > m1
        gt2 = v > m2
        nm2 = jnp.where(gt1, m1, jnp.where(gt2, v, m2))
        ni2 = jnp.where(gt1, i1, jnp.where(gt2, ev, i2))
        m1 = jnp.where(gt1, v, m1)
        i1 = jnp.where(gt1, ev, i1)
        m2 = nm2
        i2 = ni2
    return m1, i1, m2, i2


@functools.partial(
    pl.kernel,
    out_type=(
        jax.ShapeDtypeStruct((CHUNK, 2), jnp.int32),
        jax.ShapeDtypeStruct((CHUNK, 2), jnp.float32),
    ),
    mesh=_SC_MESH,
    compiler_params=pltpu.CompilerParams(needs_layout_passes=False),
    scratch_types=[
        pltpu.VMEM((1, N_EXP, BLK), jnp.float32),
        pltpu.VMEM((BLK, 2), jnp.int32),
        pltpu.VMEM((BLK, 2), jnp.float32),
    ],
)
def _route_sc(logits_hbm, idx_hbm, w_hbm, ltile, idxv, wv):
    wid = lax.axis_index("s") * 2 + lax.axis_index("c")

    lanes = lax.iota(jnp.int32, LANES)
    col0 = jnp.zeros((LANES,), jnp.int32)
    col1 = col0 + 1

    def emit(tok, m1, i1, m2, i2):
        ed = jnp.exp(m2 - m1)
        s = ed + 1.0
        plsc.store_scatter(idxv, [tok, col0], i1)
        plsc.store_scatter(idxv, [tok, col1], i2)
        plsc.store_scatter(wv, [tok, col0], 1.0 / s)
        plsc.store_scatter(wv, [tok, col1], ed / s)

    def per_pair(p, carry):
        off_a = p * (2 * LANES)
        off_b = off_a + LANES
        ra = _top2_stream(ltile, off_a)
        rb = _top2_stream(ltile, off_b)
        emit(off_a + lanes, *ra)
        emit(off_b + lanes, *rb)
        return carry

    for k in range(CHUNK // BLK // N_WORKERS):
        blk = wid + k * N_WORKERS
        pltpu.sync_copy(logits_hbm.at[pl.ds(blk, 1)], ltile)
        lax.fori_loop(0, BLK // (2 * LANES), per_pair, 0)
        pltpu.sync_copy(idxv, idx_hbm.at[pl.ds(blk * BLK, BLK), :])
        pltpu.sync_copy(wv, w_hbm.at[pl.ds(blk * BLK, BLK), :])


def kernel(hidden_states, weight):
    routed = [_route_sc(_compute_logits_blocked(hidden_states, weight, c))
              for c in range(N_CHUNKS)]
    topk_idx = jnp.concatenate([r[0] for r in routed], axis=0)
    topk_weight = jnp.concatenate([r[1] for r in routed], axis=0)
    return topk_idx, topk_weight


# speedup vs baseline: 1.0974x; 1.0006x over previous
import functools

import jax
import jax.numpy as jnp
from jax import lax
from jax.experimental import pallas as pl
from jax.experimental.pallas import tpu as pltpu
from jax.experimental.pallas import tpu_sc as plsc

N_TOK = 8192
D_MODEL = 2048
N_EXP = 64
LANES = 16
BLK = 128
N_WORKERS = 32
N_BLOCKS = N_TOK // BLK
N_CHUNKS = 2
CHUNK = N_TOK // N_CHUNKS
TOK_BLK = 1024


def _logits_body(w_ref, hs_ref, out_ref):
    lg = lax.dot_general(
        w_ref[...], hs_ref[...],
        dimension_numbers=(((1,), (1,)), ((), ())),
        preferred_element_type=jnp.float32,
    )
    for j in range(TOK_BLK // BLK):
        out_ref[j, :, :] = lg[:, j * BLK:(j + 1) * BLK]


def _compute_logits_blocked(hidden_states, weight, chunk):
    blk0 = chunk * (CHUNK // TOK_BLK)
    return pl.pallas_call(
        _logits_body,
        grid=(CHUNK // TOK_BLK,),
        in_specs=[
            pl.BlockSpec((N_EXP, D_MODEL), lambda i: (0, 0)),
            pl.BlockSpec((TOK_BLK, D_MODEL), lambda i: (blk0 + i, 0)),
        ],
        out_specs=pl.BlockSpec((TOK_BLK // BLK, N_EXP, BLK), lambda i: (i, 0, 0)),
        out_shape=jax.ShapeDtypeStruct((CHUNK // BLK, N_EXP, BLK), jnp.float32),
    )(weight, hidden_states)


_SC_MESH = plsc.VectorSubcoreMesh(core_axis_name="c", subcore_axis_name="s")


def _top2_stream(ltile, off):
    m1 = ltile[0, 0, pl.ds(off, LANES)]
    i1 = jnp.zeros((LANES,), jnp.int32)
    m2 = jnp.full((LANES,), -jnp.inf, jnp.float32)
    i2 = i1
    for e in range(1, N_EXP):
        v = ltile[0, e, pl.ds(off, LANES)]
        ev = jnp.full((LANES,), e, jnp.int32)
        gt1 = v > m1
        gt2 = v > m2
        nm2 = jnp.where(gt1, m1, jnp.where(gt2, v, m2))
        ni2 = jnp.where(gt1, i1, jnp.where(gt2, ev, i2))
        m1 = jnp.where(gt1, v, m1)
        i1 = jnp.where(gt1, ev, i1)
        m2 = nm2
        i2 = ni2
    return m1, i1, m2, i2


@functools.partial(
    pl.kernel,
    out_type=(
        jax.ShapeDtypeStruct((CHUNK, 2), jnp.int32),
        jax.ShapeDtypeStruct((CHUNK, 2), jnp.float32),
    ),
    mesh=_SC_MESH,
    compiler_params=pltpu.CompilerParams(needs_layout_passes=False, skip_device_barrier=True),
    scratch_types=[
        pltpu.VMEM((1, N_EXP, BLK), jnp.float32),
        pltpu.VMEM((BLK, 2), jnp.int32),
        pltpu.VMEM((BLK, 2), jnp.float32),
    ],
)
def _route_sc(logits_hbm, idx_hbm, w_hbm, ltile, idxv, wv):
    wid = lax.axis_index("s") * 2 + lax.axis_index("c")

    lanes = lax.iota(jnp.int32, LANES)
    col0 = jnp.zeros((LANES,), jnp.int32)
    col1 = col0 + 1

    def emit(tok, m1, i1, m2, i2):
        ed = jnp.exp(m2 - m1)
        s = ed + 1.0
        plsc.store_scatter(idxv, [tok, col0], i1)
        plsc.store_scatter(idxv, [tok, col1], i2)
        plsc.store_scatter(wv, [tok, col0], 1.0 / s)
        plsc.store_scatter(wv, [tok, col1], ed / s)

    def per_pair(p, carry):
        off_a = p * (2 * LANES)
        off_b = off_a + LANES
        ra = _top2_stream(ltile, off_a)
        rb = _top2_stream(ltile, off_b)
        emit(off_a + lanes, *ra)
        emit(off_b + lanes, *rb)
        return carry

    for k in range(CHUNK // BLK // N_WORKERS):
        blk = wid + k * N_WORKERS
        pltpu.sync_copy(logits_hbm.at[pl.ds(blk, 1)], ltile)
        lax.fori_loop(0, BLK // (2 * LANES), per_pair, 0)
        pltpu.sync_copy(idxv, idx_hbm.at[pl.ds(blk * BLK, BLK), :])
        pltpu.sync_copy(wv, w_hbm.at[pl.ds(blk * BLK, BLK), :])


def kernel(hidden_states, weight):
    routed = [_route_sc(_compute_logits_blocked(hidden_states, weight, c))
              for c in range(N_CHUNKS)]
    topk_idx = jnp.concatenate([r[0] for r in routed], axis=0)
    topk_weight = jnp.concatenate([r[1] for r in routed], axis=0)
    return topk_idx, topk_weight


# speedup vs baseline: 1.1122x; 1.0135x over previous
import functools

import jax
import jax.numpy as jnp
from jax import lax
from jax.experimental import pallas as pl
from jax.experimental.pallas import tpu as pltpu
from jax.experimental.pallas import tpu_sc as plsc

N_TOK = 8192
D_MODEL = 2048
N_EXP = 64
LANES = 16
N_WORKERS = 32
TPW = N_TOK // N_WORKERS
TOK_BLK = 1024


def _logits_body(w_ref, hs_ref, out_ref):
    out_ref[...] = lax.dot_general(
        w_ref[...], hs_ref[...],
        dimension_numbers=(((1,), (1,)), ((), ())),
        preferred_element_type=jnp.float32,
    )


def _compute_logits_t(hidden_states, weight):
    return pl.pallas_call(
        _logits_body,
        grid=(N_TOK // TOK_BLK,),
        in_specs=[
            pl.BlockSpec((N_EXP, D_MODEL), lambda i: (0, 0)),
            pl.BlockSpec((TOK_BLK, D_MODEL), lambda i: (i, 0)),
        ],
        out_specs=pl.BlockSpec((N_EXP, TOK_BLK), lambda i: (0, i)),
        out_shape=jax.ShapeDtypeStruct((N_EXP, N_TOK), jnp.float32),
    )(weight, hidden_states)


_SC_MESH = plsc.VectorSubcoreMesh(core_axis_name="c", subcore_axis_name="s")


def _top2_stream(ltile, off):
    m1 = ltile[0, pl.ds(off, LANES)]
    i1 = jnp.zeros((LANES,), jnp.int32)
    m2 = jnp.full((LANES,), -jnp.inf, jnp.float32)
    i2 = i1
    for e in range(1, N_EXP):
        v = ltile[e, pl.ds(off, LANES)]
        ev = jnp.full((LANES,), e, jnp.int32)
        gt1 = v > m1
        gt2 = v > m2
        nm2 = jnp.where(gt1, m1, jnp.where(gt2, v, m2))
        ni2 = jnp.where(gt1, i1, jnp.where(gt2, ev, i2))
        m1 = jnp.where(gt1, v, m1)
        i1 = jnp.where(gt1, ev, i1)
        m2 = nm2
        i2 = ni2
    return m1, i1, m2, i2


@functools.partial(
    pl.kernel,
    out_type=(
        jax.ShapeDtypeStruct((N_TOK, 2), jnp.int32),
        jax.ShapeDtypeStruct((N_TOK, 2), jnp.float32),
    ),
    mesh=_SC_MESH,
    compiler_params=pltpu.CompilerParams(needs_layout_passes=False),
    scratch_types=[
        pltpu.VMEM((N_EXP, TPW), jnp.float32),
        pltpu.VMEM((TPW, 2), jnp.int32),
        pltpu.VMEM((TPW, 2), jnp.float32),
    ],
)
def _route_sc(logits_hbm, idx_hbm, w_hbm, ltile, idxv, wv):
    wid = lax.axis_index("s") * 2 + lax.axis_index("c")
    base = wid * TPW
    pltpu.sync_copy(logits_hbm.at[:, pl.ds(base, TPW)], ltile)

    lanes = lax.iota(jnp.int32, LANES)
    col0 = jnp.zeros((LANES,), jnp.int32)
    col1 = col0 + 1

    def emit(tok, m1, i1, m2, i2):
        ed = jnp.exp(m2 - m1)
        s = ed + 1.0
        plsc.store_scatter(idxv, [tok, col0], i1)
        plsc.store_scatter(idxv, [tok, col1], i2)
        plsc.store_scatter(wv, [tok, col0], 1.0 / s)
        plsc.store_scatter(wv, [tok, col1], ed / s)

    def per_pair(p, carry):
        off_a = p * (2 * LANES)
        off_b = off_a + LANES
        ra = _top2_stream(ltile, off_a)
        rb = _top2_stream(ltile, off_b)
        emit(off_a + lanes, *ra)
        emit(off_b + lanes, *rb)
        return carry

    lax.fori_loop(0, TPW // (2 * LANES), per_pair, 0)

    pltpu.sync_copy(idxv, idx_hbm.at[pl.ds(base, TPW), :])
    pltpu.sync_copy(wv, w_hbm.at[pl.ds(base, TPW), :])


def kernel(hidden_states, weight):
    logits_t = _compute_logits_t(hidden_states, weight)
    topk_idx, topk_weight = _route_sc(logits_t)
    return topk_idx, topk_weight
